# Initial kernel scaffold; baseline (speedup 1.0000x reference)
import jax
import jax.numpy as jnp
from jax.experimental import pallas as pl


def kernel(x, emb_table):
    raise NotImplementedError("write your pallas kernel here")


# speedup vs baseline: 2.9869x; 2.9869x over previous
import jax
import jax.numpy as jnp
from jax.experimental import pallas as pl


def _copy_body(in_ref, out_ref):
    out_ref[...] = in_ref[...]


def kernel(x, emb_table):
    seq_len = x.shape[1]
    dim = emb_table.shape[1]
    n_blocks = 8
    block_rows = seq_len // n_blocks
    return pl.pallas_call(
        _copy_body,
        grid=(n_blocks,),
        in_specs=[pl.BlockSpec((block_rows, dim), lambda i: (i, 0))],
        out_specs=pl.BlockSpec((block_rows, dim), lambda i: (i, 0)),
        out_shape=jax.ShapeDtypeStruct((seq_len, dim), emb_table.dtype),
    )(emb_table)
